# Initial kernel scaffold; baseline (speedup 1.0000x reference)
import jax
import jax.numpy as jnp
from jax.experimental import pallas as pl


def kernel(clip1_embeddings, clip2_embeddings):
    raise NotImplementedError("write your pallas kernel here")


# speedup vs baseline: 23.9610x; 23.9610x over previous
import functools

import jax
import jax.numpy as jnp
from jax import lax
from jax.experimental import pallas as pl
from jax.experimental.pallas import tpu as pltpu
from jax.experimental.pallas import tpu_sc as plsc

B, N, D = 4, 2048, 1024
TILE = 256
NT = N // TILE
EPS = 1e-8
CHUNK = 64


def _sim_argmax_body(c1_ref, c2_ref, avg_ref, idx_ref):
    b = pl.program_id(0)
    t = pl.program_id(1)
    c1 = c1_ref[0]
    c2 = c2_ref[0]
    n2 = jnp.maximum(jnp.sqrt(jnp.sum(c2 * c2, axis=1)), EPS)
    dots = lax.dot_general(c1, c2, (((1,), (1,)), ((), ())),
                           preferred_element_type=jnp.float32)
    sim = dots / n2[None, :]
    mx = jnp.max(sim, axis=1, keepdims=True)
    ii = lax.broadcasted_iota(jnp.int32, sim.shape, 1)
    am = jnp.min(jnp.where(sim == mx, ii, N), axis=1)
    idx_ref[0, 0, 0, :] = am + b * N
    avg_ref[0] = (c1 + c2_ref[0, pl.ds(t * TILE, TILE), :]) * 0.5


def _sim_argmax(clip1, clip2, interpret=False):
    return pl.pallas_call(
        _sim_argmax_body,
        grid=(B, NT),
        in_specs=[
            pl.BlockSpec((1, TILE, D), lambda b, t: (b, t, 0)),
            pl.BlockSpec((1, N, D), lambda b, t: (b, 0, 0)),
        ],
        out_specs=[
            pl.BlockSpec((1, TILE, D), lambda b, t: (b, t, 0)),
            pl.BlockSpec((1, 1, 1, TILE), lambda b, t: (b, t, 0, 0)),
        ],
        out_shape=[
            jax.ShapeDtypeStruct((B, N, D), jnp.float32),
            jax.ShapeDtypeStruct((B, NT, 1, TILE), jnp.int32),
        ],
        interpret=interpret,
    )(clip1, clip2)


@functools.cache
def _gather_rows_kernel():
    info = plsc.get_sparse_core_info()
    nc, ns = info.num_cores, info.num_subcores
    nw = nc * ns
    rows = B * N
    rpw = rows // nw
    mesh = plsc.VectorSubcoreMesh(core_axis_name="c", subcore_axis_name="s")

    @functools.partial(
        pl.kernel,
        mesh=mesh,
        out_type=jax.ShapeDtypeStruct((rows, D), jnp.float32),
        scratch_types=[
            pltpu.VMEM((rpw,), jnp.int32),
            pltpu.VMEM((CHUNK, D), jnp.float32),
            pltpu.SemaphoreType.DMA,
        ],
    )
    def gather(avg_hbm, idx_hbm, out_hbm, idx_v, rows_v, sem):
        wid = lax.axis_index("s") * nc + lax.axis_index("c")
        base = wid * rpw
        pltpu.sync_copy(idx_hbm.at[pl.ds(base, rpw)], idx_v)

        def body(c, carry):
            pltpu.async_copy(
                avg_hbm.at[idx_v.at[pl.ds(c * CHUNK, CHUNK)]], rows_v, sem
            ).wait()
            pltpu.sync_copy(rows_v, out_hbm.at[pl.ds(base + c * CHUNK, CHUNK)])
            return carry

        lax.fori_loop(0, rpw // CHUNK, body, 0)

    return gather


def kernel(clip1_embeddings, clip2_embeddings):
    avg, idx = _sim_argmax(clip1_embeddings, clip2_embeddings)
    merged = _gather_rows_kernel()(avg.reshape(B * N, D), idx.reshape(B * N))
    return merged.reshape(B, N, D)
